# Initial kernel scaffold; baseline (speedup 1.0000x reference)
import jax
import jax.numpy as jnp
from jax.experimental import pallas as pl


def kernel(x, w1, b1, w2p, b2p):
    raise NotImplementedError("write your pallas kernel here")


# speedup vs baseline: 4.7200x; 4.7200x over previous
import jax
import jax.numpy as jnp
from jax.experimental import pallas as pl
from jax.experimental.pallas import tpu as pltpu

LANE = 128
SUBLANE = 8
TILE_B = 2048
VMEM_LIMIT_BYTES = 100 * 1024 * 1024


def _round_up(x, m):
    return (x + m - 1) // m * m


def _mlp_kernel(x_ref, w1_ref, b1_ref, w2_ref, b2_ref, o_ref):
    x = x_ref[...].astype(jnp.bfloat16)
    h = jnp.dot(x, w1_ref[...], preferred_element_type=jnp.float32)
    h = jnp.maximum(h + b1_ref[...], 0.0).astype(jnp.bfloat16)
    out = jnp.dot(h, w2_ref[...], preferred_element_type=jnp.float32)
    o_ref[...] = out + b2_ref[...]


def kernel(x, w1, b1, w2p, b2p):
    B, D = x.shape
    H = w1.shape[1]
    A = w2p.shape[1]
    A_pad = max(_round_up(A, LANE), LANE)
    if A_pad != A:
        w2p = jnp.pad(w2p, ((0, 0), (0, A_pad - A)))
        b2p = jnp.pad(b2p, ((0, 0), (0, A_pad - A)))

    w1b = w1.astype(jnp.bfloat16)
    w2b = w2p.astype(jnp.bfloat16)

    tb = min(TILE_B, _round_up(B, SUBLANE))
    B_pad = _round_up(B, tb)
    if B_pad != B:
        x = jnp.pad(x, ((0, B_pad - B), (0, 0)))
    n_tiles = B_pad // tb

    out = pl.pallas_call(
        _mlp_kernel,
        out_shape=jax.ShapeDtypeStruct((B_pad, A_pad), jnp.float32),
        grid=(n_tiles,),
        in_specs=[
            pl.BlockSpec((tb, D), lambda i: (i, 0)),
            pl.BlockSpec((D, H), lambda i: (0, 0)),
            pl.BlockSpec((1, H), lambda i: (0, 0)),
            pl.BlockSpec((H, A_pad), lambda i: (0, 0)),
            pl.BlockSpec((1, A_pad), lambda i: (0, 0)),
        ],
        out_specs=pl.BlockSpec((tb, A_pad), lambda i: (i, 0)),
        compiler_params=pltpu.CompilerParams(
            dimension_semantics=("parallel",),
            vmem_limit_bytes=VMEM_LIMIT_BYTES,
        ),
    )(x, w1b, b1, w2b, b2p)

    return out[:B, :A]


# speedup vs baseline: 5.5674x; 1.1795x over previous
import jax
import jax.numpy as jnp
from jax.experimental import pallas as pl
from jax.experimental.pallas import tpu as pltpu

LANE = 128
SUBLANE = 8
TILE_B = 4096
VMEM_LIMIT_BYTES = 100 * 1024 * 1024


def _round_up(x, m):
    return (x + m - 1) // m * m


def _mlp_kernel(x_ref, w1_ref, b1_ref, w2_ref, b2_ref, o_ref):
    x = x_ref[...].astype(jnp.bfloat16)
    h = jnp.dot(x, w1_ref[...], preferred_element_type=jnp.float32)
    h = jnp.maximum(h + b1_ref[...], 0.0).astype(jnp.bfloat16)
    out = jnp.dot(h, w2_ref[...], preferred_element_type=jnp.float32)
    o_ref[...] = out + b2_ref[...]


def kernel(x, w1, b1, w2p, b2p):
    B, D = x.shape
    H = w1.shape[1]
    A = w2p.shape[1]
    A_pad = max(_round_up(A, LANE), LANE)
    if A_pad != A:
        w2p = jnp.pad(w2p, ((0, 0), (0, A_pad - A)))
        b2p = jnp.pad(b2p, ((0, 0), (0, A_pad - A)))

    w1b = w1.astype(jnp.bfloat16)
    w2b = w2p.astype(jnp.bfloat16)

    tb = min(TILE_B, _round_up(B, SUBLANE))
    B_pad = _round_up(B, tb)
    if B_pad != B:
        x = jnp.pad(x, ((0, B_pad - B), (0, 0)))
    n_tiles = B_pad // tb

    out = pl.pallas_call(
        _mlp_kernel,
        out_shape=jax.ShapeDtypeStruct((B_pad, A_pad), jnp.float32),
        grid=(n_tiles,),
        in_specs=[
            pl.BlockSpec((tb, D), lambda i: (i, 0)),
            pl.BlockSpec((D, H), lambda i: (0, 0)),
            pl.BlockSpec((1, H), lambda i: (0, 0)),
            pl.BlockSpec((H, A_pad), lambda i: (0, 0)),
            pl.BlockSpec((1, A_pad), lambda i: (0, 0)),
        ],
        out_specs=pl.BlockSpec((tb, A_pad), lambda i: (i, 0)),
        compiler_params=pltpu.CompilerParams(
            dimension_semantics=("parallel",),
            vmem_limit_bytes=VMEM_LIMIT_BYTES,
        ),
    )(x, w1b, b1, w2b, b2p)

    return out[:B, :A]


# speedup vs baseline: 5.9706x; 1.0724x over previous
import jax
import jax.numpy as jnp
from jax.experimental import pallas as pl
from jax.experimental.pallas import tpu as pltpu

LANE = 128
SUBLANE = 8
TILE_B = 8192
VMEM_LIMIT_BYTES = 100 * 1024 * 1024


def _round_up(x, m):
    return (x + m - 1) // m * m


def _mlp_kernel(x_ref, w1_ref, b1_ref, w2_ref, b2_ref, o_ref):
    x = x_ref[...].astype(jnp.bfloat16)
    h = jnp.dot(x, w1_ref[...], preferred_element_type=jnp.float32)
    h = jnp.maximum(h + b1_ref[...], 0.0).astype(jnp.bfloat16)
    out = jnp.dot(h, w2_ref[...], preferred_element_type=jnp.float32)
    o_ref[...] = out + b2_ref[...]


def kernel(x, w1, b1, w2p, b2p):
    B, D = x.shape
    H = w1.shape[1]
    A = w2p.shape[1]
    A_pad = max(_round_up(A, LANE), LANE)
    if A_pad != A:
        w2p = jnp.pad(w2p, ((0, 0), (0, A_pad - A)))
        b2p = jnp.pad(b2p, ((0, 0), (0, A_pad - A)))

    w1b = w1.astype(jnp.bfloat16)
    w2b = w2p.astype(jnp.bfloat16)

    tb = min(TILE_B, _round_up(B, SUBLANE))
    B_pad = _round_up(B, tb)
    if B_pad != B:
        x = jnp.pad(x, ((0, B_pad - B), (0, 0)))
    n_tiles = B_pad // tb

    out = pl.pallas_call(
        _mlp_kernel,
        out_shape=jax.ShapeDtypeStruct((B_pad, A_pad), jnp.float32),
        grid=(n_tiles,),
        in_specs=[
            pl.BlockSpec((tb, D), lambda i: (i, 0)),
            pl.BlockSpec((D, H), lambda i: (0, 0)),
            pl.BlockSpec((1, H), lambda i: (0, 0)),
            pl.BlockSpec((H, A_pad), lambda i: (0, 0)),
            pl.BlockSpec((1, A_pad), lambda i: (0, 0)),
        ],
        out_specs=pl.BlockSpec((tb, A_pad), lambda i: (i, 0)),
        compiler_params=pltpu.CompilerParams(
            dimension_semantics=("parallel",),
            vmem_limit_bytes=VMEM_LIMIT_BYTES,
        ),
    )(x, w1b, b1, w2b, b2p)

    return out[:B, :A]


# speedup vs baseline: 6.3644x; 1.0659x over previous
import jax
import jax.numpy as jnp
from jax.experimental import pallas as pl
from jax.experimental.pallas import tpu as pltpu

LANE = 128
SUBLANE = 8
TILE_B = 8192
VMEM_LIMIT_BYTES = 100 * 1024 * 1024


def _round_up(x, m):
    return (x + m - 1) // m * m


def _mlp_kernel(x_ref, w1_ref, b1_ref, w2_ref, b2_ref, o_ref):
    x = x_ref[...].astype(jnp.bfloat16)
    w1 = w1_ref[...].astype(jnp.bfloat16)
    h = jnp.dot(x, w1, preferred_element_type=jnp.float32)
    h = jnp.maximum(h + b1_ref[...], 0.0).astype(jnp.bfloat16)
    w2 = w2_ref[...].astype(jnp.bfloat16)
    out = jnp.dot(h, w2, preferred_element_type=jnp.float32)
    o_ref[...] = out + b2_ref[...]


def kernel(x, w1, b1, w2p, b2p):
    B, D = x.shape
    H = w1.shape[1]
    A = w2p.shape[1]
    A_pad = max(_round_up(A, LANE), LANE)
    if A_pad != A:
        w2p = jnp.pad(w2p, ((0, 0), (0, A_pad - A)))
        b2p = jnp.pad(b2p, ((0, 0), (0, A_pad - A)))

    tb = min(TILE_B, _round_up(B, SUBLANE))
    B_pad = _round_up(B, tb)
    if B_pad != B:
        x = jnp.pad(x, ((0, B_pad - B), (0, 0)))
    n_tiles = B_pad // tb

    out = pl.pallas_call(
        _mlp_kernel,
        out_shape=jax.ShapeDtypeStruct((B_pad, A_pad), jnp.float32),
        grid=(n_tiles,),
        in_specs=[
            pl.BlockSpec((tb, D), lambda i: (i, 0)),
            pl.BlockSpec((D, H), lambda i: (0, 0)),
            pl.BlockSpec((1, H), lambda i: (0, 0)),
            pl.BlockSpec((H, A_pad), lambda i: (0, 0)),
            pl.BlockSpec((1, A_pad), lambda i: (0, 0)),
        ],
        out_specs=pl.BlockSpec((tb, A_pad), lambda i: (i, 0)),
        compiler_params=pltpu.CompilerParams(
            dimension_semantics=("parallel",),
            vmem_limit_bytes=VMEM_LIMIT_BYTES,
        ),
    )(x, w1, b1, w2p, b2p)

    return out[:B, :A]


# speedup vs baseline: 6.3732x; 1.0014x over previous
import jax
import jax.numpy as jnp
from jax.experimental import pallas as pl
from jax.experimental.pallas import tpu as pltpu

LANE = 128
SUBLANE = 8
TILE_B = 8192
VMEM_LIMIT_BYTES = 100 * 1024 * 1024


def _round_up(x, m):
    return (x + m - 1) // m * m


def _mlp_kernel(x_ref, w1_ref, b1_ref, w2_ref, b2_ref, o_ref):
    x = x_ref[...].astype(jnp.bfloat16)
    w1 = w1_ref[...].astype(jnp.bfloat16)
    h = jnp.dot(x, w1, preferred_element_type=jnp.float32)
    b1b = b1_ref[...].astype(jnp.bfloat16)
    h = jnp.maximum(h.astype(jnp.bfloat16) + b1b, jnp.bfloat16(0.0))
    w2 = w2_ref[...].astype(jnp.bfloat16)
    out = jnp.dot(h, w2, preferred_element_type=jnp.float32)
    o_ref[...] = out + b2_ref[...]


def kernel(x, w1, b1, w2p, b2p):
    B, D = x.shape
    H = w1.shape[1]
    A = w2p.shape[1]
    A_pad = max(_round_up(A, LANE), LANE)
    if A_pad != A:
        w2p = jnp.pad(w2p, ((0, 0), (0, A_pad - A)))
        b2p = jnp.pad(b2p, ((0, 0), (0, A_pad - A)))

    tb = min(TILE_B, _round_up(B, SUBLANE))
    B_pad = _round_up(B, tb)
    if B_pad != B:
        x = jnp.pad(x, ((0, B_pad - B), (0, 0)))
    n_tiles = B_pad // tb

    out = pl.pallas_call(
        _mlp_kernel,
        out_shape=jax.ShapeDtypeStruct((B_pad, A_pad), jnp.float32),
        grid=(n_tiles,),
        in_specs=[
            pl.BlockSpec((tb, D), lambda i: (i, 0)),
            pl.BlockSpec((D, H), lambda i: (0, 0)),
            pl.BlockSpec((1, H), lambda i: (0, 0)),
            pl.BlockSpec((H, A_pad), lambda i: (0, 0)),
            pl.BlockSpec((1, A_pad), lambda i: (0, 0)),
        ],
        out_specs=pl.BlockSpec((tb, A_pad), lambda i: (i, 0)),
        compiler_params=pltpu.CompilerParams(
            dimension_semantics=("parallel",),
            vmem_limit_bytes=VMEM_LIMIT_BYTES,
        ),
    )(x, w1, b1, w2p, b2p)

    return out[:B, :A]
